# baseline (speedup 1.0000x reference)
import functools
import jax
import jax.numpy as jnp
from jax import lax
from jax.experimental import pallas as pl
from jax.experimental.pallas import tpu as pltpu
from jax.experimental.pallas import tpu_sc as plsc

N_USER = 5000
N_NODES = 10000
NP = 10240
D = 128
DH = D // 2
E = 320000
L = 16
NC = 2
NS = 16
E_PER_TILE = E // NS
B = 80
NBLK = E_PER_TILE // B
CHUNKI = 25
NCHUNK = NBLK // CHUNKI
CB = CHUNKI * B
NQUAD = (CHUNKI - 2) // 4
DJ = DH // L
ROWS_PER_TILE = NP // NS
ZCHUNKS = ROWS_PER_TILE // B
NSLOT = 4

_mesh = plsc.VectorSubcoreMesh(
    core_axis_name="c", subcore_axis_name="s", num_cores=NC, num_subcores=NS)


@functools.partial(
    pl.kernel,
    out_type=jax.ShapeDtypeStruct((NP, 2, NC, DH), jnp.float32),
    mesh=_mesh,
    scratch_types=[
        pltpu.VMEM((CB,), jnp.int32),
        pltpu.VMEM((CHUNKI, B), jnp.int32),
        pltpu.VMEM((CHUNKI, B), jnp.float32),
        [pltpu.VMEM((B, DH), jnp.float32)] * NSLOT,
        pltpu.VMEM_SHARED((NP, DH), jnp.float32),
        pltpu.VMEM_SHARED((NP, DH), jnp.float32),
        [pltpu.SemaphoreType.DMA] * NSLOT,
        [pltpu.SemaphoreType.DMA] * NSLOT,
        pltpu.SemaphoreType.DMA,
    ],
    compiler_params=pltpu.CompilerParams(
        needs_layout_passes=False, use_tc_tiling_on_sc=False),
)
def _gcn2(x_hbm, dst_hbm, src_hbm, w_hbm, out_hbm,
          didx, sidx2, wbuf2, rowbufs, acc1, acc2, gsems, ssems, wsem):
    cid = lax.axis_index("c")
    sid = lax.axis_index("s")

    zeros = jnp.zeros((L,), jnp.float32)

    @pl.loop(0, B)
    def _zero(i):
        for j in range(DJ):
            rowbufs[0][i, pl.ds(j * L, L)] = zeros

    for k in range(ZCHUNKS):
        r0 = sid * ROWS_PER_TILE + k * B
        pltpu.sync_copy(rowbufs[0], acc1.at[pl.ds(r0, B)])
        pltpu.sync_copy(rowbufs[0], acc2.at[pl.ds(r0, B)])
    plsc.subcore_barrier()

    def make_layer(src_ref, acc, transform_didx):
        def issue_gather(j, s):
            pltpu.async_copy(src_ref.at[didx.at[pl.ds(j * B, B)]],
                             rowbufs[s], gsems[s])

        def wait_gather(s):
            pltpu.make_async_copy(
                x_hbm.at[pl.ds(0, B)], rowbufs[s], gsems[s]).wait()

        def issue_scatter(j, s):
            pltpu.async_copy(rowbufs[s], acc.at[sidx2.at[j]], ssems[s],
                             add=True)

        def wait_scatter(s):
            pltpu.make_async_copy(
                x_hbm.at[pl.ds(0, B)], rowbufs[s], ssems[s]).wait()

        def scale(j, s):
            rows = rowbufs[s]

            @plsc.parallel_loop(0, B, 1, unroll=8)
            def _edge(e):
                wvec = wbuf2[j, pl.ds((e // L) * L, L)]
                wsp = lax.gather(
                    wvec, jnp.full((L, 1), e % L, jnp.int32),
                    lax.GatherDimensionNumbers(
                        offset_dims=(), collapsed_slice_dims=(0,),
                        start_index_map=(0,)),
                    (1,), mode=lax.GatherScatterMode.PROMISE_IN_BOUNDS)
                for k in range(DJ):
                    rows[e, pl.ds(k * L, L)] = rows[e, pl.ds(k * L, L)] * wsp

        def step(b, s, prefetch, wait_prev_scatter=True):
            if prefetch:
                s_pre = (s + 2) % NSLOT
                if wait_prev_scatter:
                    wait_scatter(s_pre)
                issue_gather(b + 2, s_pre)
            wait_gather(s)
            scale(b, s)
            issue_scatter(b, s)

        @pl.loop(0, NCHUNK)
        def _chunk(c):
            pltpu.sync_copy(dst_hbm.at[sid, c], didx)
            pltpu.sync_copy(src_hbm.at[sid, c], sidx2)
            pltpu.sync_copy(w_hbm.at[sid, c], wbuf2)

            if transform_didx:
                cvec = jnp.full((L,), cid, jnp.int32)

                @plsc.parallel_loop(0, CB // L, 1, unroll=8)
                def _xf(i):
                    sl = pl.ds(i * L, L)
                    didx[sl] = didx[sl] * 2 + cvec

            issue_gather(0, 0)
            issue_gather(1, 1)

            step(0, 0, prefetch=True, wait_prev_scatter=False)
            step(1, 1, prefetch=True, wait_prev_scatter=False)
            step(2, 2, prefetch=True)
            step(3, 3, prefetch=True)

            @pl.loop(1, NQUAD)
            def _quad(q):
                b0 = 4 * q
                for i in range(4):
                    step(b0 + i, i, prefetch=True)

            for b in range(4 * NQUAD, CHUNKI):
                step(b, b % NSLOT, prefetch=(b + 2 < CHUNKI))
            for s in range(NSLOT):
                wait_scatter(s)

    make_layer(x_hbm, acc1, transform_didx=True)
    plsc.subcore_barrier()

    r0 = sid * ROWS_PER_TILE
    w1 = pltpu.async_copy(acc1.at[pl.ds(r0, ROWS_PER_TILE)],
                          out_hbm.at[pl.ds(r0, ROWS_PER_TILE), 0, cid], wsem)

    make_layer(acc1, acc2, transform_didx=False)
    plsc.subcore_barrier()

    pltpu.sync_copy(acc2.at[pl.ds(r0, ROWS_PER_TILE)],
                    out_hbm.at[pl.ds(r0, ROWS_PER_TILE), 1, cid])
    w1.wait()


@jax.jit
def kernel(ini_embeds, edge_index, adj_values):
    src = edge_index[0].astype(jnp.int32).reshape(NS, NCHUNK, CHUNKI, B)
    dst = edge_index[1].astype(jnp.int32).reshape(NS, NCHUNK, CB)
    w = adj_values.reshape(NS, NCHUNK, CHUNKI, B)

    x2 = ini_embeds.reshape(2 * N_NODES, DH)

    out = _gcn2(x2, dst, src, w)
    tem = out.reshape(NP, 2 * D)[:N_NODES]
    return tem[:N_USER], tem[N_USER:]
